# Initial kernel scaffold; baseline (speedup 1.0000x reference)
import jax
import jax.numpy as jnp
from jax.experimental import pallas as pl


def kernel(patches):
    raise NotImplementedError("write your pallas kernel here")


# speedup vs baseline: 3.3386x; 3.3386x over previous
import functools

import jax
import jax.numpy as jnp
from jax import lax
from jax.experimental import pallas as pl
from jax.experimental.pallas import tpu as pltpu
from jax.experimental.pallas import tpu_sc as plsc

RATIO = 0.75


@functools.lru_cache(maxsize=None)
def _make_gather(num_rows, C, NC, NS, n_chunks, chunk):
    NW = NC * NS
    mesh = plsc.VectorSubcoreMesh(core_axis_name="c", subcore_axis_name="s")

    @functools.partial(
        pl.kernel,
        mesh=mesh,
        out_type=jax.ShapeDtypeStruct((num_rows, C), jnp.float32),
        scratch_types=[
            pltpu.VMEM((n_chunks, chunk), jnp.int32),
            pltpu.VMEM((chunk, C), jnp.float32),
            pltpu.SemaphoreType.DMA,
        ],
    )
    def gather_k(table_hbm, idx_hbm, out_hbm, idx_v, rows_v, sem):
        wid = lax.axis_index("s") * NC + lax.axis_index("c")
        pltpu.sync_copy(idx_hbm.at[wid], idx_v)
        base = wid * (n_chunks * chunk)
        for c in range(n_chunks):
            pltpu.async_copy(table_hbm.at[idx_v.at[c]], rows_v, sem).wait()
            pltpu.sync_copy(rows_v, out_hbm.at[pl.ds(base + c * chunk, chunk)])

    return gather_k


def kernel(patches):
    T, B, C = patches.shape
    remain_T = int(T * (1 - RATIO))
    perm_key = jax.random.key(42)
    keys = jax.random.split(perm_key, B)
    fwd = jnp.stack([jax.random.permutation(k, T) for k in keys], axis=-1)
    bwd = jnp.argsort(fwd, axis=0)

    src = fwd[:remain_T] * B + jnp.arange(B, dtype=jnp.int32)[None, :]
    num_rows = remain_T * B

    info = plsc.get_sparse_core_info()
    NC, NS = info.num_cores, info.num_subcores
    NW = NC * NS
    rows_per_w = num_rows // NW
    assert rows_per_w * NW == num_rows
    chunk = 96
    n_chunks = rows_per_w // chunk
    assert n_chunks * chunk == rows_per_w

    idx3 = src.reshape(NW, n_chunks, chunk).astype(jnp.int32)
    table = patches.reshape(T * B, C)
    out_flat = _make_gather(num_rows, C, NC, NS, n_chunks, chunk)(table, idx3)
    return out_flat.reshape(remain_T, B, C), fwd, bwd
